# baseline (speedup 1.0000x reference)
import functools

import jax
import jax.numpy as jnp
from jax import lax
from jax.experimental import pallas as pl
from jax.experimental.pallas import tpu as pltpu
from jax.experimental.pallas import tpu_sc as plsc

_B, _S, _H, _E, _F = 1, 2048, 2048, 8, 768
_T = _B * _S
_P = 2 * _T
_RT = 256
_NTILES = _P // _RT + _E
_NSLOT = _NTILES * _RT
_PC = 512
_NW = 32


def _router_perm_body(hs_ref, gw_ref, logits_ref, xb_ref, w0_ref, w1_ref,
                      d0_ref, d1_ref, stok_ref, te_ref):
    x = hs_ref[...]
    xb_ref[...] = x.astype(jnp.bfloat16)
    logits = lax.dot_general(x, gw_ref[...], (((1,), (1,)), ((), ())),
                             preferred_element_type=jnp.float32)
    logits_ref[...] = logits
    p = jax.nn.softmax(logits, axis=-1)
    eio = lax.broadcasted_iota(jnp.int32, (_T, _E), 1)
    m1 = jnp.max(p, axis=-1, keepdims=True)
    i1 = jnp.argmax(p, axis=-1)[:, None]
    oh1 = (eio == i1).astype(jnp.float32)
    p2 = jnp.where(oh1 > 0, -jnp.inf, p)
    m2 = jnp.max(p2, axis=-1, keepdims=True)
    i2 = jnp.argmax(p2, axis=-1)[:, None]
    oh2 = (eio == i2).astype(jnp.float32)
    wsum = m1 + m2
    w0_ref[...] = m1 / wsum
    w1_ref[...] = m2 / wsum

    oh = jnp.concatenate([oh1, oh2], axis=0)
    sio = lax.broadcasted_iota(jnp.int32, (_PC, _PC), 0)
    lio = lax.broadcasted_iota(jnp.int32, (_PC, _PC), 1)
    tri = (lio <= sio).astype(jnp.float32)
    nchunk = _P // _PC
    cums = []
    run = jnp.zeros((1, _E), jnp.float32)
    for c in range(nchunk):
        blk = oh[c * _PC:(c + 1) * _PC, :]
        cum = jnp.dot(tri, blk, preferred_element_type=jnp.float32) + run
        cums.append(cum)
        run = cum[_PC - 1:_PC, :]
    cum_all = jnp.concatenate(cums, axis=0)
    counts_l = run
    rank = jnp.sum(cum_all * oh, axis=-1, keepdims=True) - 1.0

    padded = jnp.floor((counts_l + float(_RT - 1)) / float(_RT)) * float(_RT)
    e_sio = lax.broadcasted_iota(jnp.int32, (_E, _E), 0)
    e_lio = lax.broadcasted_iota(jnp.int32, (_E, _E), 1)
    tri8_strict = (e_sio < e_lio).astype(jnp.float32)
    off_l = jnp.dot(padded, tri8_strict,
                    preferred_element_type=jnp.float32)
    off_pair = jnp.sum(oh * off_l, axis=-1, keepdims=True)
    dest = off_pair + rank
    d0_ref[...] = dest[:_T].astype(jnp.int32)
    d1_ref[...] = dest[_T:].astype(jnp.int32)

    tok = jnp.concatenate(
        [lax.broadcasted_iota(jnp.int32, (_T, 1), 0).astype(jnp.float32)] * 2,
        axis=0)
    slot_sio = lax.broadcasted_iota(jnp.int32, (_PC, 1), 0)
    for c in range(_NSLOT // _PC):
        dl = jnp.transpose(dest)
        ohT = (dl.astype(jnp.int32) == (slot_sio + c * _PC)
               ).astype(jnp.float32)
        st = jnp.dot(ohT, tok, preferred_element_type=jnp.float32)
        stok_ref[c * _PC:(c + 1) * _PC, :] = st.astype(jnp.int32)

    ends_l = off_l + padded
    ts = (lax.broadcasted_iota(jnp.int32, (1, _NTILES), 1).astype(jnp.float32)
          * float(_RT))
    offs = jnp.transpose(off_l)
    ends = jnp.transpose(ends_l)
    m = jnp.logical_and(offs <= ts, ts < ends).astype(jnp.float32)
    started = (ends <= ts).astype(jnp.float32)
    te = jnp.sum(started, axis=0, keepdims=True)
    active = jnp.sum(m, axis=0, keepdims=True)
    te = jnp.minimum(te, float(_E - 1))
    te_ref[...] = jnp.concatenate([te, active], axis=0).astype(jnp.int32)


def _router_perm(hs, gate_w):
    return pl.pallas_call(
        _router_perm_body,
        in_specs=[
            pl.BlockSpec((_T, _H), lambda: (0, 0)),
            pl.BlockSpec((_E, _H), lambda: (0, 0)),
        ],
        out_specs=[
            pl.BlockSpec((_T, _E), lambda: (0, 0)),
            pl.BlockSpec((_T, _H), lambda: (0, 0)),
            pl.BlockSpec((_T, 1), lambda: (0, 0)),
            pl.BlockSpec((_T, 1), lambda: (0, 0)),
            pl.BlockSpec((_T, 1), lambda: (0, 0)),
            pl.BlockSpec((_T, 1), lambda: (0, 0)),
            pl.BlockSpec((_NSLOT, 1), lambda: (0, 0)),
            pl.BlockSpec((2, _NTILES), lambda: (0, 0)),
        ],
        out_shape=[
            jax.ShapeDtypeStruct((_T, _E), jnp.float32),
            jax.ShapeDtypeStruct((_T, _H), jnp.bfloat16),
            jax.ShapeDtypeStruct((_T, 1), jnp.float32),
            jax.ShapeDtypeStruct((_T, 1), jnp.float32),
            jax.ShapeDtypeStruct((_T, 1), jnp.int32),
            jax.ShapeDtypeStruct((_T, 1), jnp.int32),
            jax.ShapeDtypeStruct((_NSLOT, 1), jnp.int32),
            jax.ShapeDtypeStruct((2, _NTILES), jnp.int32),
        ],
    )(hs, gate_w)


def _gather_rows(table, idx):
    b = idx.shape[0]
    bw = b // _NW
    ch = 64
    mesh = plsc.VectorSubcoreMesh(core_axis_name="c", subcore_axis_name="s")

    @functools.partial(
        pl.kernel, mesh=mesh,
        out_type=jax.ShapeDtypeStruct((b, 8, 128), jnp.int32),
        scratch_types=[
            pltpu.VMEM((bw,), jnp.int32),
            pltpu.VMEM((ch, 8, 128), jnp.int32),
            pltpu.SemaphoreType.DMA,
        ],
    )
    def k(table_hbm, idx_hbm, out_hbm, idx_v, rows_v, sem):
        wid = lax.axis_index("s") * 2 + lax.axis_index("c")
        base = wid * bw
        pltpu.sync_copy(idx_hbm.at[pl.ds(base, bw)], idx_v)
        for c in range(bw // ch):
            pltpu.async_copy(
                table_hbm.at[idx_v.at[pl.ds(c * ch, ch)]], rows_v, sem).wait()
            pltpu.sync_copy(rows_v, out_hbm.at[pl.ds(base + c * ch, ch)])

    return k(table, idx)


def _grouped_ffn_body(te_ref, xs_ref, w1_ref, w2_ref, y_ref, w1b_ref, w2b_ref):
    rt = pl.program_id(0)
    e_now = te_ref[0, rt]
    e_prev = te_ref[0, jnp.maximum(rt - 1, 0)]
    changed = jnp.logical_or(rt == 0, e_now != e_prev)

    @pl.when(changed)
    def _():
        w1b_ref[...] = w1_ref[0].astype(jnp.bfloat16)
        w2b_ref[...] = w2_ref[0].astype(jnp.bfloat16)

    @pl.when(te_ref[1, rt] == 1)
    def _():
        x = xs_ref[...]
        gu = jnp.dot(x, w1b_ref[...], preferred_element_type=jnp.float32)
        g = gu[:, :_F]
        u = gu[:, _F:]
        inter = (u * (g * jax.nn.sigmoid(g))).astype(jnp.bfloat16)
        y_ref[...] = jnp.dot(
            inter, w2b_ref[...],
            preferred_element_type=jnp.float32).astype(jnp.bfloat16)


def _grouped_ffn(te, xs, gate_up_proj, down_proj):
    grid_spec = pltpu.PrefetchScalarGridSpec(
        num_scalar_prefetch=1,
        grid=(_NTILES,),
        in_specs=[
            pl.BlockSpec((_RT, _H), lambda rt, s: (rt, 0)),
            pl.BlockSpec((1, _H, 2 * _F), lambda rt, s: (s[0, rt], 0, 0)),
            pl.BlockSpec((1, _F, _H), lambda rt, s: (s[0, rt], 0, 0)),
        ],
        out_specs=pl.BlockSpec((_RT, _H), lambda rt, s: (rt, 0)),
        scratch_shapes=[
            pltpu.VMEM((_H, 2 * _F), jnp.bfloat16),
            pltpu.VMEM((_F, _H), jnp.bfloat16),
        ],
    )
    return pl.pallas_call(
        _grouped_ffn_body,
        grid_spec=grid_spec,
        out_shape=jax.ShapeDtypeStruct((_NSLOT, _H), jnp.bfloat16),
        compiler_params=pltpu.CompilerParams(
            dimension_semantics=("arbitrary",)),
    )(te, xs, gate_up_proj, down_proj)


def _combine_body(y0_ref, y1_ref, w0_ref, w1_ref, out_ref):
    y0 = y0_ref[...].astype(jnp.float32)
    y1 = y1_ref[...].astype(jnp.float32)
    out_ref[...] = w0_ref[...] * y0 + w1_ref[...] * y1


def _combine(y0, y1, w0, w1):
    tt = 512
    return pl.pallas_call(
        _combine_body,
        grid=(_T // tt,),
        in_specs=[
            pl.BlockSpec((tt, _H), lambda t: (t, 0)),
            pl.BlockSpec((tt, _H), lambda t: (t, 0)),
            pl.BlockSpec((tt, 1), lambda t: (t, 0)),
            pl.BlockSpec((tt, 1), lambda t: (t, 0)),
        ],
        out_specs=pl.BlockSpec((tt, _H), lambda t: (t, 0)),
        out_shape=jax.ShapeDtypeStruct((_T, _H), jnp.float32),
    )(y0, y1, w0, w1)


def _as_i32_rows(a, n):
    return lax.bitcast_convert_type(
        a.reshape(n, _H // 2, 2), jnp.int32).reshape(n, 8, 128)


def _as_bf16_rows(a, n):
    return lax.bitcast_convert_type(
        a.reshape(n, _H // 2), jnp.bfloat16).reshape(n, _H)


def kernel(hidden_states, gate_w, gate_up_proj, down_proj):
    hs = hidden_states.reshape(_T, _H)
    (logits, xb, w0, w1, d0, d1, stok, te) = _router_perm(hs, gate_w)
    xs = _gather_rows(_as_i32_rows(xb, _T), stok.reshape(_NSLOT))
    y = _grouped_ffn(te, _as_bf16_rows(xs, _NSLOT), gate_up_proj, down_proj)
    y3 = _as_i32_rows(y, _NSLOT)
    y0 = _gather_rows(y3, d0.reshape(_T))
    y1 = _gather_rows(y3, d1.reshape(_T))
    out = _combine(_as_bf16_rows(y0, _T), _as_bf16_rows(y1, _T), w0, w1)
    return out.reshape(_B, _S, _H), logits


# speedup vs baseline: 4.1323x; 4.1323x over previous
import jax
import jax.numpy as jnp
from jax.experimental import pallas as pl
from jax.experimental.pallas import tpu as pltpu

_B, _S, _H, _E, _F = 1, 2048, 2048, 8, 768
_FH = 384
_TT = 512
_HT = 256
_RT = 512


def _router_body(hs_ref, gw_ref, logits_ref, wd_ref, xb_ref):
    x = hs_ref[...]
    xb_ref[...] = x.astype(jnp.bfloat16)
    logits = jax.lax.dot_general(
        x, gw_ref[...], (((1,), (1,)), ((), ())),
        preferred_element_type=jnp.float32)
    logits_ref[...] = logits
    p = jax.nn.softmax(logits, axis=-1)
    eio = jax.lax.broadcasted_iota(jnp.int32, p.shape, 1)
    m1 = jnp.max(p, axis=-1, keepdims=True)
    i1 = jnp.argmax(p, axis=-1)[:, None]
    oh1 = eio == i1
    p2 = jnp.where(oh1, -jnp.inf, p)
    m2 = jnp.max(p2, axis=-1, keepdims=True)
    i2 = jnp.argmax(p2, axis=-1)[:, None]
    oh2 = eio == i2
    wd_ref[...] = (jnp.where(oh1, m1, 0.0)
                   + jnp.where(oh2, m2, 0.0)) / (m1 + m2)


def _expert_body(xb_ref, wd_ref, wg_ref, wu_ref, w2_ref, out_ref,
                 wgb_ref, wub_ref, w2b_ref):
    e = pl.program_id(0)
    f = pl.program_id(1)
    t = pl.program_id(2)

    @pl.when(t == 0)
    def _():
        wgb_ref[...] = wg_ref[0].astype(jnp.bfloat16)
        wub_ref[...] = wu_ref[0].astype(jnp.bfloat16)
        w2b_ref[...] = w2_ref[0].astype(jnp.bfloat16)

    @pl.when(jnp.logical_and(jnp.logical_and(e == 0, f == 0), t == 0))
    def _():
        out_ref[...] = jnp.zeros_like(out_ref)
    eio = jax.lax.broadcasted_iota(jnp.int32, (_HT, _E), 1)
    wgb = wgb_ref[...]
    wub = wub_ref[...]
    w2b = w2b_ref[...]

    row_sl = [pl.ds(t * _TT + h * _HT, _HT) for h in range(_TT // _HT)]
    xs = [xb_ref[r, :] for r in row_sl]
    gs = [jnp.dot(x, wgb, preferred_element_type=jnp.float32) for x in xs]
    us = [jnp.dot(x, wub, preferred_element_type=jnp.float32) for x in xs]
    contribs = []
    for h, r in enumerate(row_sl):
        we = jnp.sum(jnp.where(eio == e, wd_ref[r, :], 0.0),
                     axis=-1, keepdims=True)
        g, u = gs[h], us[h]
        inter = (we * (u * (g * jax.nn.sigmoid(g)))).astype(jnp.bfloat16)
        contribs.append(
            jnp.dot(inter, w2b, preferred_element_type=jnp.float32))
    for h, r in enumerate(row_sl):
        out_ref[r, :] += contribs[h]


def kernel(hidden_states, gate_w, gate_up_proj, down_proj):
    T = _B * _S
    hs = hidden_states.reshape(T, _H)
    logits, wdense, xb = pl.pallas_call(
        _router_body,
        grid=(T // _RT,),
        in_specs=[
            pl.BlockSpec((_RT, _H), lambda t: (t, 0)),
            pl.BlockSpec((_E, _H), lambda t: (0, 0)),
        ],
        out_specs=[
            pl.BlockSpec((_RT, _E), lambda t: (t, 0)),
            pl.BlockSpec((_RT, _E), lambda t: (t, 0)),
            pl.BlockSpec((_RT, _H), lambda t: (t, 0)),
        ],
        out_shape=[
            jax.ShapeDtypeStruct((T, _E), jnp.float32),
            jax.ShapeDtypeStruct((T, _E), jnp.float32),
            jax.ShapeDtypeStruct((T, _H), jnp.bfloat16),
        ],
    )(hs, gate_w)

    out = pl.pallas_call(
        _expert_body,
        grid=(_E, _F // _FH, T // _TT),
        in_specs=[
            pl.BlockSpec((T, _H), lambda e, f, t: (0, 0)),
            pl.BlockSpec((T, _E), lambda e, f, t: (0, 0)),
            pl.BlockSpec((1, _H, _FH), lambda e, f, t: (e, 0, f)),
            pl.BlockSpec((1, _H, _FH), lambda e, f, t: (e, 0, f + _F // _FH)),
            pl.BlockSpec((1, _FH, _H), lambda e, f, t: (e, f, 0)),
        ],
        out_specs=pl.BlockSpec((T, _H), lambda e, f, t: (0, 0)),
        out_shape=jax.ShapeDtypeStruct((T, _H), jnp.float32),
        scratch_shapes=[
            pltpu.VMEM((_H, _FH), jnp.bfloat16),
            pltpu.VMEM((_H, _FH), jnp.bfloat16),
            pltpu.VMEM((_FH, _H), jnp.bfloat16),
        ],
        compiler_params=pltpu.CompilerParams(
            dimension_semantics=("arbitrary", "arbitrary", "arbitrary")),
    )(xb, wdense, gate_up_proj, gate_up_proj, down_proj)
    return out.reshape(_B, _S, _H), logits


# speedup vs baseline: 4.3243x; 1.0465x over previous
import jax
import jax.numpy as jnp
from jax.experimental import pallas as pl
from jax.experimental.pallas import tpu as pltpu

_B, _S, _H, _E, _F = 1, 2048, 2048, 8, 768
_FH = 384
_TT = 1024
_HT = 256
_RT = 512


def _router_body(hs_ref, gw_ref, logits_ref, wd_ref, xb_ref):
    x = hs_ref[...]
    xb_ref[...] = x.astype(jnp.bfloat16)
    logits = jax.lax.dot_general(
        x, gw_ref[...], (((1,), (1,)), ((), ())),
        preferred_element_type=jnp.float32)
    logits_ref[...] = logits
    p = jax.nn.softmax(logits, axis=-1)
    eio = jax.lax.broadcasted_iota(jnp.int32, p.shape, 1)
    m1 = jnp.max(p, axis=-1, keepdims=True)
    i1 = jnp.argmax(p, axis=-1)[:, None]
    oh1 = eio == i1
    p2 = jnp.where(oh1, -jnp.inf, p)
    m2 = jnp.max(p2, axis=-1, keepdims=True)
    i2 = jnp.argmax(p2, axis=-1)[:, None]
    oh2 = eio == i2
    wd_ref[...] = (jnp.where(oh1, m1, 0.0)
                   + jnp.where(oh2, m2, 0.0)) / (m1 + m2)


def _expert_body(xb_ref, wd_ref, wg_ref, wu_ref, w2_ref, out_ref,
                 wgb_ref, wub_ref, w2b_ref):
    e = pl.program_id(0)
    f = pl.program_id(1)
    t = pl.program_id(2)

    @pl.when(t == 0)
    def _():
        wgb_ref[...] = wg_ref[0].astype(jnp.bfloat16)
        wub_ref[...] = wu_ref[0].astype(jnp.bfloat16)
        w2b_ref[...] = w2_ref[0].astype(jnp.bfloat16)

    @pl.when(jnp.logical_and(jnp.logical_and(e == 0, f == 0), t == 0))
    def _():
        out_ref[...] = jnp.zeros_like(out_ref)
    eio = jax.lax.broadcasted_iota(jnp.int32, (_HT, _E), 1)
    wgb = wgb_ref[...]
    wub = wub_ref[...]
    w2b = w2b_ref[...]

    row_sl = [pl.ds(t * _TT + h * _HT, _HT) for h in range(_TT // _HT)]
    xs = [xb_ref[r, :] for r in row_sl]
    gs = [jnp.dot(x, wgb, preferred_element_type=jnp.float32) for x in xs]
    us = [jnp.dot(x, wub, preferred_element_type=jnp.float32) for x in xs]
    contribs = []
    for h, r in enumerate(row_sl):
        we = jnp.sum(jnp.where(eio == e, wd_ref[r, :], 0.0),
                     axis=-1, keepdims=True)
        g, u = gs[h], us[h]
        inter = (we * (u * (g * jax.nn.sigmoid(g)))).astype(jnp.bfloat16)
        contribs.append(
            jnp.dot(inter, w2b, preferred_element_type=jnp.float32))
    for h, r in enumerate(row_sl):
        out_ref[r, :] += contribs[h]


def kernel(hidden_states, gate_w, gate_up_proj, down_proj):
    T = _B * _S
    hs = hidden_states.reshape(T, _H)
    logits, wdense, xb = pl.pallas_call(
        _router_body,
        grid=(T // _RT,),
        in_specs=[
            pl.BlockSpec((_RT, _H), lambda t: (t, 0)),
            pl.BlockSpec((_E, _H), lambda t: (0, 0)),
        ],
        out_specs=[
            pl.BlockSpec((_RT, _E), lambda t: (t, 0)),
            pl.BlockSpec((_RT, _E), lambda t: (t, 0)),
            pl.BlockSpec((_RT, _H), lambda t: (t, 0)),
        ],
        out_shape=[
            jax.ShapeDtypeStruct((T, _E), jnp.float32),
            jax.ShapeDtypeStruct((T, _E), jnp.float32),
            jax.ShapeDtypeStruct((T, _H), jnp.bfloat16),
        ],
    )(hs, gate_w)

    out = pl.pallas_call(
        _expert_body,
        grid=(_E, _F // _FH, T // _TT),
        in_specs=[
            pl.BlockSpec((T, _H), lambda e, f, t: (0, 0)),
            pl.BlockSpec((T, _E), lambda e, f, t: (0, 0)),
            pl.BlockSpec((1, _H, _FH), lambda e, f, t: (e, 0, f)),
            pl.BlockSpec((1, _H, _FH), lambda e, f, t: (e, 0, f + _F // _FH)),
            pl.BlockSpec((1, _FH, _H), lambda e, f, t: (e, f, 0)),
        ],
        out_specs=pl.BlockSpec((T, _H), lambda e, f, t: (0, 0)),
        out_shape=jax.ShapeDtypeStruct((T, _H), jnp.float32),
        scratch_shapes=[
            pltpu.VMEM((_H, _FH), jnp.bfloat16),
            pltpu.VMEM((_H, _FH), jnp.bfloat16),
            pltpu.VMEM((_FH, _H), jnp.bfloat16),
        ],
        compiler_params=pltpu.CompilerParams(
            dimension_semantics=("arbitrary", "arbitrary", "arbitrary")),
    )(xb, wdense, gate_up_proj, gate_up_proj, down_proj)
    return out.reshape(_B, _S, _H), logits


# speedup vs baseline: 4.4434x; 1.0275x over previous
import jax
import jax.numpy as jnp
from jax.experimental import pallas as pl
from jax.experimental.pallas import tpu as pltpu

_B, _S, _H, _E, _F = 1, 2048, 2048, 8, 768
_FH = 384
_TT = 2048
_HT = 512
_RT = 512


def _router_body(hs_ref, gw_ref, logits_ref, wd_ref, xb_ref):
    x = hs_ref[...]
    xb_ref[...] = x.astype(jnp.bfloat16)
    logits = jax.lax.dot_general(
        x, gw_ref[...], (((1,), (1,)), ((), ())),
        preferred_element_type=jnp.float32)
    logits_ref[...] = logits
    p = jax.nn.softmax(logits, axis=-1)
    eio = jax.lax.broadcasted_iota(jnp.int32, p.shape, 1)
    m1 = jnp.max(p, axis=-1, keepdims=True)
    i1 = jnp.argmax(p, axis=-1)[:, None]
    oh1 = eio == i1
    p2 = jnp.where(oh1, -jnp.inf, p)
    m2 = jnp.max(p2, axis=-1, keepdims=True)
    i2 = jnp.argmax(p2, axis=-1)[:, None]
    oh2 = eio == i2
    wd_ref[...] = (jnp.where(oh1, m1, 0.0)
                   + jnp.where(oh2, m2, 0.0)) / (m1 + m2)


def _expert_body(xb_ref, wd_ref, wg_ref, wu_ref, w2_ref, out_ref,
                 wgb_ref, wub_ref, w2b_ref):
    e = pl.program_id(0)
    f = pl.program_id(1)
    t = pl.program_id(2)

    @pl.when(t == 0)
    def _():
        wgb_ref[...] = wg_ref[0].astype(jnp.bfloat16)
        wub_ref[...] = wu_ref[0].astype(jnp.bfloat16)
        w2b_ref[...] = w2_ref[0].astype(jnp.bfloat16)

    @pl.when(jnp.logical_and(jnp.logical_and(e == 0, f == 0), t == 0))
    def _():
        out_ref[...] = jnp.zeros_like(out_ref)
    eio = jax.lax.broadcasted_iota(jnp.int32, (_HT, _E), 1)
    wgb = wgb_ref[...]
    wub = wub_ref[...]
    w2b = w2b_ref[...]

    row_sl = [pl.ds(t * _TT + h * _HT, _HT) for h in range(_TT // _HT)]
    xs = [xb_ref[r, :] for r in row_sl]
    gs = [jnp.dot(x, wgb, preferred_element_type=jnp.float32) for x in xs]
    us = [jnp.dot(x, wub, preferred_element_type=jnp.float32) for x in xs]
    contribs = []
    for h, r in enumerate(row_sl):
        we = jnp.sum(jnp.where(eio == e, wd_ref[r, :], 0.0),
                     axis=-1, keepdims=True)
        g, u = gs[h], us[h]
        inter = (we * (u * (g * jax.nn.sigmoid(g)))).astype(jnp.bfloat16)
        contribs.append(
            jnp.dot(inter, w2b, preferred_element_type=jnp.float32))
    for h, r in enumerate(row_sl):
        out_ref[r, :] += contribs[h]


def kernel(hidden_states, gate_w, gate_up_proj, down_proj):
    T = _B * _S
    hs = hidden_states.reshape(T, _H)
    logits, wdense, xb = pl.pallas_call(
        _router_body,
        grid=(T // _RT,),
        in_specs=[
            pl.BlockSpec((_RT, _H), lambda t: (t, 0)),
            pl.BlockSpec((_E, _H), lambda t: (0, 0)),
        ],
        out_specs=[
            pl.BlockSpec((_RT, _E), lambda t: (t, 0)),
            pl.BlockSpec((_RT, _E), lambda t: (t, 0)),
            pl.BlockSpec((_RT, _H), lambda t: (t, 0)),
        ],
        out_shape=[
            jax.ShapeDtypeStruct((T, _E), jnp.float32),
            jax.ShapeDtypeStruct((T, _E), jnp.float32),
            jax.ShapeDtypeStruct((T, _H), jnp.bfloat16),
        ],
    )(hs, gate_w)

    out = pl.pallas_call(
        _expert_body,
        grid=(_E, _F // _FH, T // _TT),
        in_specs=[
            pl.BlockSpec((T, _H), lambda e, f, t: (0, 0)),
            pl.BlockSpec((T, _E), lambda e, f, t: (0, 0)),
            pl.BlockSpec((1, _H, _FH), lambda e, f, t: (e, 0, f)),
            pl.BlockSpec((1, _H, _FH), lambda e, f, t: (e, 0, f + _F // _FH)),
            pl.BlockSpec((1, _FH, _H), lambda e, f, t: (e, f, 0)),
        ],
        out_specs=pl.BlockSpec((T, _H), lambda e, f, t: (0, 0)),
        out_shape=jax.ShapeDtypeStruct((T, _H), jnp.float32),
        scratch_shapes=[
            pltpu.VMEM((_H, _FH), jnp.bfloat16),
            pltpu.VMEM((_H, _FH), jnp.bfloat16),
            pltpu.VMEM((_FH, _H), jnp.bfloat16),
        ],
        compiler_params=pltpu.CompilerParams(
            dimension_semantics=("arbitrary", "arbitrary", "arbitrary")),
    )(xb, wdense, gate_up_proj, gate_up_proj, down_proj)
    return out.reshape(_B, _S, _H), logits
